# baseline (speedup 1.0000x reference)
import functools

import jax
import jax.numpy as jnp
from jax.experimental import pallas as pl
from jax.experimental.pallas import tpu as pltpu

DIM = 64
VOCAB1 = 1000001
BLOCK = 8192
K = 8
STEP = BLOCK * K
NBLK = -(-VOCAB1 // STEP)


def _lse_kernel(batch_ref, u_ref, *refs):
    v_refs = refs[:K]
    out_ref = refs[K]
    acc_ref = refs[K + 1]
    i = pl.program_id(0)

    @pl.when(i == 0)
    def _init():
        acc_ref[0] = -jnp.inf
        acc_ref[1] = 0.0
        acc_ref[2] = 0.0

    u = u_ref[...].reshape(1, DIM)
    y = batch_ref[1]

    m_old = acc_ref[0]
    bmax = m_old
    bsum = 0.0
    zy = 0.0
    zs = []
    for b in range(K):
        v = v_refs[b][...]
        z = jax.lax.dot_general(
            u, v, (((1,), (0,)), ((), ())), preferred_element_type=jnp.float32
        )
        col = (i * K + b) * BLOCK + jax.lax.broadcasted_iota(
            jnp.int32, (1, BLOCK), 1
        )
        z = jnp.where(col < VOCAB1, z, -jnp.inf)
        zy += jnp.sum(jnp.where(col == y, z, 0.0))
        bmax = jnp.maximum(bmax, jnp.max(z))
        zs.append(z)
    for z in zs:
        bsum += jnp.sum(jnp.exp(z - bmax))

    acc_ref[2] += zy
    acc_ref[1] = acc_ref[1] * jnp.exp(m_old - bmax) + bsum
    acc_ref[0] = bmax

    @pl.when(i == NBLK - 1)
    def _finish():
        out_ref[0, 0] = (jnp.log(acc_ref[1]) + acc_ref[0]) - acc_ref[2]


LAST_VBLK = (VOCAB1 - 1) // BLOCK


def _v_spec(b):
    return pl.BlockSpec(
        (DIM, BLOCK),
        lambda i, bb, b=b: (0, jnp.minimum(i * K + b, LAST_VBLK)),
    )


@jax.jit
def _skipgram_loss(batch, u_table, v_table):
    grid_spec = pltpu.PrefetchScalarGridSpec(
        num_scalar_prefetch=1,
        grid=(NBLK,),
        in_specs=[pl.BlockSpec((1, 1, DIM), lambda i, b: (b[0], 0, 0))]
        + [_v_spec(b) for b in range(K)],
        out_specs=pl.BlockSpec(memory_space=pltpu.SMEM),
        scratch_shapes=[pltpu.SMEM((4,), jnp.float32)],
    )
    out = pl.pallas_call(
        _lse_kernel,
        grid_spec=grid_spec,
        out_shape=jax.ShapeDtypeStruct((1, 1), jnp.float32),
    )(batch.astype(jnp.int32), u_table.reshape(-1, 1, DIM), *([v_table] * K))
    return out[0, 0]


def kernel(batch, u_table, v_table):
    return _skipgram_loss(batch, u_table, v_table)


# speedup vs baseline: 2.6354x; 2.6354x over previous
import functools

import jax
import jax.numpy as jnp
from jax.experimental import pallas as pl
from jax.experimental.pallas import tpu as pltpu

DIM = 64
VOCAB1 = 1000001
BLOCK = 8192
K = 8
STEP = BLOCK * K
NBLK = -(-VOCAB1 // STEP)


def _lse_kernel(batch_ref, u_ref, *refs):
    v_refs = refs[:K]
    out_ref = refs[K]
    acc_ref = refs[K + 1]
    i = pl.program_id(0)

    @pl.when(i == 0)
    def _init():
        acc_ref[0] = -jnp.inf
        acc_ref[1] = 0.0
        acc_ref[2] = 0.0

    r = batch_ref[0] % 8
    u8 = u_ref[...]
    row = jax.lax.broadcasted_iota(jnp.int32, (8, DIM), 0)
    u = jnp.sum(jnp.where(row == r, u8, 0.0), axis=0, keepdims=True)
    y = batch_ref[1]

    m_old = acc_ref[0]
    bmax = m_old
    bsum = 0.0
    zy = 0.0
    zs = []
    for b in range(K):
        v = v_refs[b][...]
        z = jax.lax.dot_general(
            u, v, (((1,), (0,)), ((), ())), preferred_element_type=jnp.float32
        )
        col = (i * K + b) * BLOCK + jax.lax.broadcasted_iota(
            jnp.int32, (1, BLOCK), 1
        )
        z = jnp.where(col < VOCAB1, z, -jnp.inf)
        zy += jnp.sum(jnp.where(col == y, z, 0.0))
        bmax = jnp.maximum(bmax, jnp.max(z))
        zs.append(z)
    for z in zs:
        bsum += jnp.sum(jnp.exp(z - bmax))

    acc_ref[2] += zy
    acc_ref[1] = acc_ref[1] * jnp.exp(m_old - bmax) + bsum
    acc_ref[0] = bmax

    @pl.when(i == NBLK - 1)
    def _finish():
        out_ref[0, 0] = (jnp.log(acc_ref[1]) + acc_ref[0]) - acc_ref[2]


LAST_VBLK = (VOCAB1 - 1) // BLOCK


def _v_spec(b):
    return pl.BlockSpec(
        (DIM, BLOCK),
        lambda i, bb, b=b: (0, jnp.minimum(i * K + b, LAST_VBLK)),
    )


@jax.jit
def _skipgram_loss(batch, u_table, v_table):
    grid_spec = pltpu.PrefetchScalarGridSpec(
        num_scalar_prefetch=1,
        grid=(NBLK,),
        in_specs=[pl.BlockSpec((8, DIM), lambda i, b: (b[0] // 8, 0))]
        + [_v_spec(b) for b in range(K)],
        out_specs=pl.BlockSpec(memory_space=pltpu.SMEM),
        scratch_shapes=[pltpu.SMEM((4,), jnp.float32)],
    )
    out = pl.pallas_call(
        _lse_kernel,
        grid_spec=grid_spec,
        out_shape=jax.ShapeDtypeStruct((1, 1), jnp.float32),
    )(batch.astype(jnp.int32), u_table, *([v_table] * K))
    return out[0, 0]


def kernel(batch, u_table, v_table):
    return _skipgram_loss(batch, u_table, v_table)
